# Initial kernel scaffold; baseline (speedup 1.0000x reference)
import jax
import jax.numpy as jnp
from jax.experimental import pallas as pl
from jax.experimental.pallas import tpu as pltpu
from jax.experimental.pallas import tpu_sc as plsc

_INTERPRET = False


def _argmin_body(zp_ref, cbt_ref, out_ref):
    m = pl.program_id(1)
    zp = zp_ref[...]
    cbt = cbt_ref[0]
    mt = zp.shape[0]
    zp2 = jnp.sum(zp * zp, axis=1, keepdims=True)
    c2 = jnp.sum(cbt * cbt, axis=0, keepdims=True)
    mm = jnp.dot(zp, cbt, preferred_element_type=jnp.float32)
    dists = zp2 - 2.0 * mm + c2
    minval = jnp.min(dists, axis=1, keepdims=True)
    iota = jax.lax.broadcasted_iota(jnp.int32, dists.shape, 1)
    idx = jnp.min(jnp.where(dists == minval, iota, jnp.int32(2 ** 30)), axis=1)
    out_ref[0, 0, pl.ds(m * mt, mt)] = idx.astype(jnp.int32)


def _compute_indices(z_flat, cbt_all):
    n, d = z_flat.shape
    n_cb, dpc, v = cbt_all.shape
    mt = min(256, n)
    out = pl.pallas_call(
        _argmin_body,
        grid=(n_cb, n // mt),
        in_specs=[
            pl.BlockSpec((mt, dpc), lambda i, m: (m, i)),
            pl.BlockSpec((1, dpc, v), lambda i, m: (i, 0, 0)),
        ],
        out_specs=pl.BlockSpec((1, 1, n), lambda i, m: (i, 0, 0)),
        out_shape=jax.ShapeDtypeStruct((n_cb, 1, n), jnp.int32),
        interpret=_INTERPRET,
    )(z_flat, cbt_all)
    return out.reshape(n_cb, n)


def _sc_gather(cb_flat, gidx):
    n_idx = gidx.shape[0]
    val_dim = cb_flat.shape[1]
    gw = 128
    mesh = plsc.VectorSubcoreMesh(core_axis_name="c", subcore_axis_name="s")
    idx2 = gidx.reshape(1, n_idx)

    @pl.kernel(
        out_type=jax.ShapeDtypeStruct((n_idx, val_dim), cb_flat.dtype),
        mesh=mesh,
    )
    def gather_kernel(x_hbm, i_hbm, o_hbm):
        def body(i_vmem, o_vmem):
            pltpu.sync_copy(x_hbm.at[i_vmem.at[0]], o_vmem)

        pltpu.emit_pipeline(
            body,
            grid=(n_idx // gw,),
            in_specs=[pl.BlockSpec((1, gw), index_map=lambda i: (0, i))],
            out_specs=[pl.BlockSpec((gw, val_dim), index_map=lambda i: (i, 0))],
            core_axis_name=("c", "s"),
            dimension_semantics=(pltpu.PARALLEL,),
        )(i_hbm, o_hbm)

    return gather_kernel(cb_flat, idx2)


def _commit_body(z_ref, q_ref, out_ref):
    @pl.when(pl.program_id(0) == 0)
    def _():
        out_ref[0, 0] = jnp.float32(0.0)

    diff = z_ref[...] - q_ref[...]
    out_ref[0, 0] += jnp.sum(diff * diff)


def _commitment(z_flat, q_flat):
    n, d = z_flat.shape
    bt = min(1024, n)
    out = pl.pallas_call(
        _commit_body,
        grid=(n // bt,),
        in_specs=[
            pl.BlockSpec((bt, d), lambda m: (m, 0)),
            pl.BlockSpec((bt, d), lambda m: (m, 0)),
        ],
        out_specs=pl.BlockSpec((1, 1), lambda m: (0, 0)),
        out_shape=jax.ShapeDtypeStruct((1, 1), jnp.float32),
        interpret=_INTERPRET,
    )(z_flat, q_flat)
    return (out / jnp.float32(z_flat.size))[0, 0]


def kernel(z, codebooks):
    b, d, h, w = z.shape
    n_cb, v, dpc = codebooks.shape
    n = b * h * w
    z_flat = jnp.transpose(z, (0, 2, 3, 1)).reshape(n, d)
    cbt = jnp.transpose(codebooks, (0, 2, 1))
    idx = _compute_indices(z_flat, cbt)
    indices = idx.reshape(n_cb, b, h, w).transpose(1, 0, 2, 3)
    gidx = (idx.T + (jnp.arange(n_cb, dtype=jnp.int32) * v)[None, :]).reshape(-1)
    q_rows = _sc_gather(codebooks.reshape(n_cb * v, dpc), gidx)
    q_flat = q_rows.reshape(n, d)
    quantized = q_flat.reshape(b, h, w, d).transpose(0, 3, 1, 2)
    commitment = _commitment(z_flat, q_flat)
    return quantized, indices, commitment


# speedup vs baseline: 1.4034x; 1.4034x over previous
import jax
import jax.numpy as jnp
from jax.experimental import pallas as pl
from jax.experimental.pallas import tpu as pltpu
from jax.experimental.pallas import tpu_sc as plsc

_INTERPRET = False


def _argmin_body(zpt_ref, cbt_ref, out_ref):
    m = pl.program_id(1)
    zpt = zpt_ref[0]
    cbt = cbt_ref[0]
    mt = zpt.shape[1]
    zp2 = jnp.transpose(jnp.sum(zpt * zpt, axis=0, keepdims=True))
    c2 = jnp.sum(cbt * cbt, axis=0, keepdims=True)
    mm = jax.lax.dot_general(
        zpt, cbt, (((0,), (0,)), ((), ())),
        preferred_element_type=jnp.float32)
    dists = zp2 - 2.0 * mm + c2
    minval = jnp.min(dists, axis=1, keepdims=True)
    iota = jax.lax.broadcasted_iota(jnp.int32, dists.shape, 1)
    idx = jnp.min(jnp.where(dists == minval, iota, jnp.int32(2 ** 30)), axis=1)
    out_ref[0, 0, pl.ds(m * mt, mt)] = idx.astype(jnp.int32)


def _compute_indices(z_t, cbt_all):
    n_cb, dpc, n = z_t.shape
    v = cbt_all.shape[2]
    mt = min(256, n)
    out = pl.pallas_call(
        _argmin_body,
        grid=(n_cb, n // mt),
        in_specs=[
            pl.BlockSpec((1, dpc, mt), lambda i, m: (i, 0, m)),
            pl.BlockSpec((1, dpc, v), lambda i, m: (i, 0, 0)),
        ],
        out_specs=pl.BlockSpec((1, 1, n), lambda i, m: (i, 0, 0)),
        out_shape=jax.ShapeDtypeStruct((n_cb, 1, n), jnp.int32),
        interpret=_INTERPRET,
    )(z_t, cbt_all)
    return out.reshape(n_cb, n)


def _sc_gather(cb_flat, gidx):
    rows, dpc = cb_flat.shape
    cb_pad = jnp.pad(cb_flat, ((0, 0), (0, 128 - dpc)))
    out_pad = _sc_gather_rows(cb_pad, gidx)
    return out_pad[:, :dpc]


def _sc_gather_rows(cb_flat, gidx):
    n_idx = gidx.shape[0]
    val_dim = cb_flat.shape[1]
    gw = 128
    mesh = plsc.VectorSubcoreMesh(core_axis_name="c", subcore_axis_name="s")
    idx2 = gidx.reshape(1, n_idx)

    @pl.kernel(
        out_type=jax.ShapeDtypeStruct((n_idx, val_dim), cb_flat.dtype),
        mesh=mesh,
    )
    def gather_kernel(x_hbm, i_hbm, o_hbm):
        def body(i_vmem, o_vmem):
            pltpu.sync_copy(x_hbm.at[i_vmem.at[0]], o_vmem)

        pltpu.emit_pipeline(
            body,
            grid=(n_idx // gw,),
            in_specs=[pl.BlockSpec((1, gw), index_map=lambda i: (0, i))],
            out_specs=[pl.BlockSpec((gw, val_dim), index_map=lambda i: (i, 0))],
            core_axis_name=("c", "s"),
            dimension_semantics=(pltpu.PARALLEL,),
        )(i_hbm, o_hbm)

    return gather_kernel(cb_flat, idx2)


def _commit_body(z_ref, q_ref, out_ref):
    @pl.when(pl.program_id(0) == 0)
    def _():
        out_ref[...] = jnp.zeros((1, 1), jnp.float32)

    diff = z_ref[...] - q_ref[...]
    out_ref[...] += jnp.sum(diff * diff).reshape(1, 1)


def _commitment(z_flat, q_flat):
    n, d = z_flat.shape
    bt = min(1024, n)
    out = pl.pallas_call(
        _commit_body,
        grid=(n // bt,),
        in_specs=[
            pl.BlockSpec((bt, d), lambda m: (m, 0)),
            pl.BlockSpec((bt, d), lambda m: (m, 0)),
        ],
        out_specs=pl.BlockSpec((1, 1), lambda m: (0, 0)),
        out_shape=jax.ShapeDtypeStruct((1, 1), jnp.float32),
        interpret=_INTERPRET,
    )(z_flat, q_flat)
    return (out / jnp.float32(z_flat.size))[0, 0]


def kernel(z, codebooks):
    b, d, h, w = z.shape
    n_cb, v, dpc = codebooks.shape
    n = b * h * w
    z_t = jnp.transpose(z, (1, 0, 2, 3)).reshape(n_cb, dpc, n)
    z_flat = jnp.transpose(z, (0, 2, 3, 1)).reshape(n, d)
    cbt = jnp.transpose(codebooks, (0, 2, 1))
    idx = _compute_indices(z_t, cbt)
    indices = idx.reshape(n_cb, b, h, w).transpose(1, 0, 2, 3)
    gidx = (idx.T + (jnp.arange(n_cb, dtype=jnp.int32) * v)[None, :]).reshape(-1)
    q_rows = _sc_gather(codebooks.reshape(n_cb * v, dpc), gidx)
    q_flat = q_rows.reshape(n, d)
    quantized = q_flat.reshape(b, h, w, d).transpose(0, 3, 1, 2)
    commitment = _commitment(z_flat, q_flat)
    return quantized, indices, commitment


# speedup vs baseline: 1.8557x; 1.3223x over previous
import jax
import jax.numpy as jnp
from jax.experimental import pallas as pl
from jax.experimental.pallas import tpu as pltpu
from jax.experimental.pallas import tpu_sc as plsc

_INTERPRET = False


def _c2_body(cbt_ref, out_ref):
    cbt = cbt_ref[0]
    out_ref[0] = jnp.sum(cbt * cbt, axis=0, keepdims=True)


def _argmin_body(zpt_ref, cbt_ref, c2_ref, out_ref):
    m = pl.program_id(1)
    zpt = zpt_ref[0]
    cbt = cbt_ref[0]
    mt = zpt.shape[1]
    zp2 = jnp.transpose(jnp.sum(zpt * zpt, axis=0, keepdims=True))
    c2 = c2_ref[0]
    mm2 = jax.lax.dot_general(
        zpt + zpt, cbt, (((0,), (0,)), ((), ())),
        preferred_element_type=jnp.float32)
    dists = zp2 - mm2 + c2
    idx = jnp.argmin(dists, axis=1)
    out_ref[0, 0, pl.ds(m * mt, mt)] = idx.astype(jnp.int32)


def _compute_indices(z_t, cbt_all):
    n_cb, dpc, n = z_t.shape
    v = cbt_all.shape[2]
    c2_all = pl.pallas_call(
        _c2_body,
        grid=(n_cb,),
        in_specs=[pl.BlockSpec((1, dpc, v), lambda i: (i, 0, 0))],
        out_specs=pl.BlockSpec((1, 1, v), lambda i: (i, 0, 0)),
        out_shape=jax.ShapeDtypeStruct((n_cb, 1, v), jnp.float32),
        interpret=_INTERPRET,
    )(cbt_all)
    mt = min(512, n)
    out = pl.pallas_call(
        _argmin_body,
        grid=(n_cb, n // mt),
        in_specs=[
            pl.BlockSpec((1, dpc, mt), lambda i, m: (i, 0, m)),
            pl.BlockSpec((1, dpc, v), lambda i, m: (i, 0, 0)),
            pl.BlockSpec((1, 1, v), lambda i, m: (i, 0, 0)),
        ],
        out_specs=pl.BlockSpec((1, 1, n), lambda i, m: (i, 0, 0)),
        out_shape=jax.ShapeDtypeStruct((n_cb, 1, n), jnp.int32),
        interpret=_INTERPRET,
    )(z_t, cbt_all, c2_all)
    return out.reshape(n_cb, n)


def _sc_gather(cb_flat, gidx):
    rows, dpc = cb_flat.shape
    cb_pad = jnp.pad(cb_flat, ((0, 0), (0, 128 - dpc)))
    out_pad = _sc_gather_rows(cb_pad, gidx)
    return out_pad[:, :dpc]


def _sc_gather_rows(cb_flat, gidx):
    n_idx = gidx.shape[0]
    val_dim = cb_flat.shape[1]
    gw = 128
    mesh = plsc.VectorSubcoreMesh(core_axis_name="c", subcore_axis_name="s")
    idx2 = gidx.reshape(1, n_idx)

    @pl.kernel(
        out_type=jax.ShapeDtypeStruct((n_idx, val_dim), cb_flat.dtype),
        mesh=mesh,
    )
    def gather_kernel(x_hbm, i_hbm, o_hbm):
        def body(i_vmem, o_vmem):
            pltpu.sync_copy(x_hbm.at[i_vmem.at[0]], o_vmem)

        pltpu.emit_pipeline(
            body,
            grid=(n_idx // gw,),
            in_specs=[pl.BlockSpec((1, gw), index_map=lambda i: (0, i))],
            out_specs=[pl.BlockSpec((gw, val_dim), index_map=lambda i: (i, 0))],
            core_axis_name=("c", "s"),
            dimension_semantics=(pltpu.PARALLEL,),
        )(i_hbm, o_hbm)

    return gather_kernel(cb_flat, idx2)


def _commit_body(z_ref, q_ref, out_ref):
    @pl.when(pl.program_id(0) == 0)
    def _():
        out_ref[...] = jnp.zeros((1, 1), jnp.float32)

    diff = z_ref[...] - q_ref[...]
    out_ref[...] += jnp.sum(diff * diff).reshape(1, 1)


def _commitment(z_flat, q_flat):
    n, d = z_flat.shape
    bt = min(1024, n)
    out = pl.pallas_call(
        _commit_body,
        grid=(n // bt,),
        in_specs=[
            pl.BlockSpec((bt, d), lambda m: (m, 0)),
            pl.BlockSpec((bt, d), lambda m: (m, 0)),
        ],
        out_specs=pl.BlockSpec((1, 1), lambda m: (0, 0)),
        out_shape=jax.ShapeDtypeStruct((1, 1), jnp.float32),
        interpret=_INTERPRET,
    )(z_flat, q_flat)
    return (out / jnp.float32(z_flat.size))[0, 0]


def kernel(z, codebooks):
    b, d, h, w = z.shape
    n_cb, v, dpc = codebooks.shape
    n = b * h * w
    z_t = jnp.transpose(z, (1, 0, 2, 3)).reshape(n_cb, dpc, n)
    z_flat = jnp.transpose(z, (0, 2, 3, 1)).reshape(n, d)
    cbt = jnp.transpose(codebooks, (0, 2, 1))
    idx = _compute_indices(z_t, cbt)
    indices = idx.reshape(n_cb, b, h, w).transpose(1, 0, 2, 3)
    gidx = (idx.T + (jnp.arange(n_cb, dtype=jnp.int32) * v)[None, :]).reshape(-1)
    q_rows = _sc_gather(codebooks.reshape(n_cb * v, dpc), gidx)
    q_flat = q_rows.reshape(n, d)
    quantized = q_flat.reshape(b, h, w, d).transpose(0, 3, 1, 2)
    commitment = _commitment(z_flat, q_flat)
    return quantized, indices, commitment


# speedup vs baseline: 1.9003x; 1.0241x over previous
import jax
import jax.numpy as jnp
from jax.experimental import pallas as pl
from jax.experimental.pallas import tpu as pltpu
from jax.experimental.pallas import tpu_sc as plsc

_INTERPRET = False


def _c2_body(cbt_ref, out_ref):
    cbt = cbt_ref[0]
    out_ref[0] = jnp.sum(cbt * cbt, axis=0, keepdims=True)


def _argmin_body(zpt_ref, cbt_ref, c2_ref, out_ref):
    m = pl.program_id(1)
    zpt = zpt_ref[0, 0]
    cbt = cbt_ref[0]
    mt = zpt.shape[1]
    zp2 = jnp.transpose(jnp.sum(zpt * zpt, axis=0, keepdims=True))
    c2 = c2_ref[0]
    mm2 = jax.lax.dot_general(
        zpt + zpt, cbt, (((0,), (0,)), ((), ())),
        preferred_element_type=jnp.float32)
    dists = zp2 - mm2 + c2
    idx = jnp.argmin(dists, axis=1)
    out_ref[0, 0, pl.ds(m * mt, mt)] = idx.astype(jnp.int32)


def _compute_indices(z4, cbt_all):
    b, n_cb, dpc, hw = z4.shape
    n = b * hw
    v = cbt_all.shape[2]
    c2_all = pl.pallas_call(
        _c2_body,
        grid=(n_cb,),
        in_specs=[pl.BlockSpec((1, dpc, v), lambda i: (i, 0, 0))],
        out_specs=pl.BlockSpec((1, 1, v), lambda i: (i, 0, 0)),
        out_shape=jax.ShapeDtypeStruct((n_cb, 1, v), jnp.float32),
        interpret=_INTERPRET,
    )(cbt_all)
    mt = min(512, hw)
    mpb = hw // mt
    out = pl.pallas_call(
        _argmin_body,
        grid=(n_cb, n // mt),
        in_specs=[
            pl.BlockSpec((1, 1, dpc, mt), lambda i, m: (m // mpb, i, 0, m % mpb)),
            pl.BlockSpec((1, dpc, v), lambda i, m: (i, 0, 0)),
            pl.BlockSpec((1, 1, v), lambda i, m: (i, 0, 0)),
        ],
        out_specs=pl.BlockSpec((1, 1, n), lambda i, m: (i, 0, 0)),
        out_shape=jax.ShapeDtypeStruct((n_cb, 1, n), jnp.int32),
        interpret=_INTERPRET,
    )(z4, cbt_all, c2_all)
    return out.reshape(n_cb, n)


def _sc_gather(cb_flat, gidx):
    rows, dpc = cb_flat.shape
    cb_pad = jnp.pad(cb_flat, ((0, 0), (0, 128 - dpc)))
    return _sc_gather_rows(cb_pad, gidx)


def _sc_gather_rows(cb_flat, gidx):
    n_idx = gidx.shape[0]
    val_dim = cb_flat.shape[1]
    gw = 128
    mesh = plsc.VectorSubcoreMesh(core_axis_name="c", subcore_axis_name="s")
    idx2 = gidx.reshape(1, n_idx)

    @pl.kernel(
        out_type=jax.ShapeDtypeStruct((n_idx, val_dim), cb_flat.dtype),
        mesh=mesh,
    )
    def gather_kernel(x_hbm, i_hbm, o_hbm):
        def body(i_vmem, o_vmem):
            pltpu.sync_copy(x_hbm.at[i_vmem.at[0]], o_vmem)

        pltpu.emit_pipeline(
            body,
            grid=(n_idx // gw,),
            in_specs=[pl.BlockSpec((1, gw), index_map=lambda i: (0, i))],
            out_specs=[pl.BlockSpec((gw, val_dim), index_map=lambda i: (i, 0))],
            core_axis_name=("c", "s"),
            dimension_semantics=(pltpu.PARALLEL,),
        )(i_hbm, o_hbm)

    return gather_kernel(cb_flat, idx2)


def _pack_commit_body(pad_ref, z_ref, q_ref, acc_ref):
    @pl.when(pl.program_id(0) == 0)
    def _():
        acc_ref[...] = jnp.zeros((1, 1), jnp.float32)

    x = pad_ref[0]
    hw = z_ref.shape[2]
    n_cb = x.shape[0] // hw
    dpc = z_ref.shape[1] // n_cb
    x3 = x.reshape(hw, n_cb, x.shape[1])
    parts = [jnp.transpose(x3[:, i, :dpc]) for i in range(n_cb)]
    qt = jnp.concatenate(parts, axis=0)
    q_ref[0] = qt
    dz = z_ref[0] - qt
    acc_ref[...] += jnp.sum(dz * dz).reshape(1, 1)


def _pack_commit(out_pad, z3):
    b, d, hw = z3.shape
    n_cb = out_pad.shape[0] // (b * hw)
    pad3 = out_pad.reshape(b, n_cb * hw, out_pad.shape[1])
    q3, acc = pl.pallas_call(
        _pack_commit_body,
        grid=(b,),
        in_specs=[
            pl.BlockSpec((1, n_cb * hw, out_pad.shape[1]), lambda m: (m, 0, 0)),
            pl.BlockSpec((1, d, hw), lambda m: (m, 0, 0)),
        ],
        out_specs=[
            pl.BlockSpec((1, d, hw), lambda m: (m, 0, 0)),
            pl.BlockSpec((1, 1), lambda m: (0, 0)),
        ],
        out_shape=[
            jax.ShapeDtypeStruct((b, d, hw), jnp.float32),
            jax.ShapeDtypeStruct((1, 1), jnp.float32),
        ],
        interpret=_INTERPRET,
    )(pad3, z3)
    return q3, (acc / jnp.float32(z3.size))[0, 0]


def kernel(z, codebooks):
    b, d, h, w = z.shape
    n_cb, v, dpc = codebooks.shape
    n = b * h * w
    hw = h * w
    z4 = z.reshape(b, n_cb, dpc, hw)
    cbt = jnp.transpose(codebooks, (0, 2, 1))
    idx = _compute_indices(z4, cbt)
    indices = idx.reshape(n_cb, b, h, w).transpose(1, 0, 2, 3)
    gidx = (idx.T + (jnp.arange(n_cb, dtype=jnp.int32) * v)[None, :]).reshape(-1)
    out_pad = _sc_gather(codebooks.reshape(n_cb * v, dpc), gidx)
    q3, commitment = _pack_commit(out_pad, z.reshape(b, d, hw))
    quantized = q3.reshape(b, d, h, w)
    return quantized, indices, commitment
